# Initial kernel scaffold; baseline (speedup 1.0000x reference)
import jax
import jax.numpy as jnp
from jax.experimental import pallas as pl


def kernel(x, edge_index, W, w):
    raise NotImplementedError("write your pallas kernel here")


# speedup vs baseline: 3.1087x; 3.1087x over previous
import functools

import jax
import jax.numpy as jnp
from jax import lax
from jax.experimental import pallas as pl
from jax.experimental.pallas import tpu as pltpu
from jax.experimental.pallas import tpu_sc as plsc

_NC = 2
_NS = 16
_NW = _NC * _NS
_LANES = 16
_EPS = 1e-5


def _u_body(xT_ref, Wt_ref, u_ref):
    u_ref[...] = jnp.dot(xT_ref[...], Wt_ref[...],
                         preferred_element_type=jnp.float32)


def _node_transform(xT, Wt, blk=1000):
    n, f = xT.shape
    return pl.pallas_call(
        _u_body,
        grid=(n // blk,),
        in_specs=[pl.BlockSpec((blk, f), lambda i: (i, 0)),
                  pl.BlockSpec((f, f), lambda i: (0, 0))],
        out_specs=pl.BlockSpec((blk, f), lambda i: (i, 0)),
        out_shape=jax.ShapeDtypeStruct((n, f), jnp.float32),
    )(xT, Wt)


def _make_finalize(num_edges):
    def _finalize_body(sp_ref, qp_ref, w_ref, a_ref, b_ref):
        wv = w_ref[0, 0]
        s = jnp.sum(sp_ref[...], axis=0, keepdims=True)
        q = jnp.sum(qp_ref[...], axis=0, keepdims=True)
        m = s / num_edges
        var = (wv * wv) * (q / num_edges - m * m)
        rstd = 1.0 / jnp.sqrt(var + _EPS)
        a_ref[...] = wv * rstd
        b_ref[...] = -(wv * m) * rstd
    return _finalize_body


def _finalize(sp, qp, w_arr, num_edges):
    nw, f = sp.shape
    return pl.pallas_call(
        _make_finalize(num_edges),
        in_specs=[pl.BlockSpec((nw, f), lambda: (0, 0)),
                  pl.BlockSpec((nw, f), lambda: (0, 0)),
                  pl.BlockSpec(memory_space=pltpu.SMEM)],
        out_specs=[pl.BlockSpec((1, f), lambda: (0, 0)),
                   pl.BlockSpec((1, f), lambda: (0, 0))],
        out_shape=[jax.ShapeDtypeStruct((1, f), jnp.float32),
                   jax.ShapeDtypeStruct((1, f), jnp.float32)],
    )(sp, qp, w_arr)


def _out_body(r_ref, w2_ref, wv_ref, o_ref):
    wv = wv_ref[0, 0]
    o_ref[...] = wv * jnp.dot(r_ref[...], w2_ref[...],
                              preferred_element_type=jnp.float32)


def _node_output(r_full, W, w_arr, blk=1000):
    n, f = r_full.shape
    return pl.pallas_call(
        _out_body,
        grid=(n // blk,),
        in_specs=[pl.BlockSpec((blk, f), lambda i: (i, 0)),
                  pl.BlockSpec((f, f), lambda i: (0, 0)),
                  pl.BlockSpec(memory_space=pltpu.SMEM)],
        out_specs=pl.BlockSpec((blk, f), lambda i: (i, 0)),
        out_shape=jax.ShapeDtypeStruct((n, f), jnp.float32),
    )(r_full, W, w_arr)


def _make_stats(n, f, e, ch):
    per_w = e // _NW
    n_ch = per_w // ch
    ng = f // _LANES
    mesh = plsc.VectorSubcoreMesh(core_axis_name="c", subcore_axis_name="s")

    @functools.partial(
        pl.kernel,
        mesh=mesh,
        out_type=[jax.ShapeDtypeStruct((_NW * f,), jnp.float32),
                  jax.ShapeDtypeStruct((_NW * f,), jnp.float32)],
        scratch_types=[
            pltpu.VMEM((ch,), jnp.int32),
            pltpu.VMEM((ch,), jnp.int32),
            pltpu.VMEM((ch, f), jnp.float32),
            pltpu.VMEM((ch, f), jnp.float32),
            pltpu.VMEM((f,), jnp.float32),
            pltpu.VMEM((f,), jnp.float32),
            pltpu.SemaphoreType.DMA,
            pltpu.SemaphoreType.DMA,
        ],
    )
    def stats(u_hbm, ii_hbm, jj_hbm, s_out, q_out,
              ii_v, jj_v, gi_v, gj_v, s_v, q_v, sem1, sem2):
        cid = lax.axis_index("c")
        sid = lax.axis_index("s")
        wid = sid * _NC + cid
        base = wid * per_w

        def chunk_body(c, carry):
            off = base + c * ch
            pltpu.sync_copy(ii_hbm.at[pl.ds(off, ch)], ii_v)
            pltpu.sync_copy(jj_hbm.at[pl.ds(off, ch)], jj_v)
            cp1 = pltpu.async_copy(u_hbm.at[ii_v], gi_v, sem1)
            cp2 = pltpu.async_copy(u_hbm.at[jj_v], gj_v, sem2)
            cp1.wait()
            cp2.wait()

            def edge_body(ei, acc):
                sa, qa = acc
                new_s = []
                new_q = []
                for g in range(ng):
                    sl = pl.ds(g * _LANES, _LANES)
                    d = gi_v[ei, sl] - gj_v[ei, sl]
                    new_s.append(sa[g] + d)
                    new_q.append(qa[g] + d * d)
                return (tuple(new_s), tuple(new_q))

            return lax.fori_loop(0, ch, edge_body, carry)

        zero = jnp.zeros((_LANES,), jnp.float32)
        init = (tuple(zero for _ in range(ng)), tuple(zero for _ in range(ng)))
        s_fin, q_fin = lax.fori_loop(0, n_ch, chunk_body, init)
        for g in range(ng):
            sl = pl.ds(g * _LANES, _LANES)
            s_v[sl] = s_fin[g]
            q_v[sl] = q_fin[g]
        pltpu.sync_copy(s_v, s_out.at[pl.ds(wid * f, f)])
        pltpu.sync_copy(q_v, q_out.at[pl.ds(wid * f, f)])

    return stats


def _make_edge_pass(n, f, e, ch):
    fh = f // _NC
    per_t = e // _NS
    n_ch = per_t // ch
    ng = fh // _LANES
    cp_rows = 200
    n_cp_total = n // cp_rows
    n_cp_iters = -(-n_cp_total // _NS)
    mesh = plsc.VectorSubcoreMesh(core_axis_name="c", subcore_axis_name="s")

    @functools.partial(
        pl.kernel,
        mesh=mesh,
        out_type=jax.ShapeDtypeStruct((_NC * n, fh), jnp.float32),
        scratch_types=[
            pltpu.VMEM((ch,), jnp.int32),
            pltpu.VMEM((ch,), jnp.int32),
            pltpu.VMEM((ch,), jnp.int32),
            pltpu.VMEM((ch,), jnp.int32),
            pltpu.VMEM((ch, fh), jnp.float32),
            pltpu.VMEM((ch, fh), jnp.float32),
            pltpu.VMEM((ch, fh), jnp.float32),
            pltpu.VMEM((ch, fh), jnp.float32),
            pltpu.VMEM((f,), jnp.float32),
            pltpu.VMEM((f,), jnp.float32),
            pltpu.VMEM((cp_rows, fh), jnp.float32),
            pltpu.VMEM_SHARED((n, fh), jnp.float32),
            pltpu.SemaphoreType.DMA,
            pltpu.SemaphoreType.DMA,
        ],
        compiler_params=pltpu.CompilerParams(use_tc_tiling_on_sc=False),
    )
    def edge_pass(u2_hbm, ii_hbm, jj_hbm, a_hbm, b_hbm, r_out,
                  ii_v, jj_v, iig_v, jjg_v, gi_v, gj_v, r_v, rn_v,
                  a_v, b_v, cp_v, racc, sem1, sem2):
        cid = lax.axis_index("c")
        sid = lax.axis_index("s")
        base = sid * per_t

        zero = jnp.zeros((_LANES,), jnp.float32)

        def zero_row(r, carry):
            for g in range(ng):
                cp_v[r, pl.ds(g * _LANES, _LANES)] = zero
            return carry

        lax.fori_loop(0, cp_rows, zero_row, 0)
        for k in range(n_cp_iters):
            cidx = k * _NS + sid

            @pl.when(cidx < n_cp_total)
            def _zero_chunk():
                pltpu.sync_copy(
                    cp_v, racc.at[pl.ds(cidx * cp_rows, cp_rows)])
        plsc.subcore_barrier()

        pltpu.sync_copy(a_hbm, a_v)
        pltpu.sync_copy(b_hbm, b_v)
        is_lo = cid == 0
        a_regs = []
        b_regs = []
        for g in range(ng):
            lo = pl.ds(g * _LANES, _LANES)
            hi = pl.ds(fh + g * _LANES, _LANES)
            a_regs.append(jnp.where(is_lo, a_v[lo], a_v[hi]))
            b_regs.append(jnp.where(is_lo, b_v[lo], b_v[hi]))

        goff = cid * n

        def chunk_body(c, carry):
            off = base + c * ch
            pltpu.sync_copy(ii_hbm.at[pl.ds(off, ch)], ii_v)
            pltpu.sync_copy(jj_hbm.at[pl.ds(off, ch)], jj_v)
            for g in range(ch // _LANES):
                sl = pl.ds(g * _LANES, _LANES)
                iig_v[sl] = ii_v[sl] + goff
                jjg_v[sl] = jj_v[sl] + goff
            cp1 = pltpu.async_copy(u2_hbm.at[iig_v], gi_v, sem1)
            cp2 = pltpu.async_copy(u2_hbm.at[jjg_v], gj_v, sem2)
            cp1.wait()
            cp2.wait()

            def edge_body(ei, acc):
                for g in range(ng):
                    sl = pl.ds(g * _LANES, _LANES)
                    d = gi_v[ei, sl] - gj_v[ei, sl]
                    t = a_regs[g] * d + b_regs[g]
                    r = jnp.maximum(t, 0.0)
                    r_v[ei, sl] = r
                    rn_v[ei, sl] = zero - r
                return acc

            lax.fori_loop(0, ch, edge_body, 0)
            pltpu.sync_copy(r_v, racc.at[ii_v], add=True)
            pltpu.sync_copy(rn_v, racc.at[jj_v], add=True)
            return carry

        lax.fori_loop(0, n_ch, chunk_body, 0)
        plsc.subcore_barrier()

        for k in range(n_cp_iters):
            cidx = k * _NS + sid

            @pl.when(cidx < n_cp_total)
            def _dump_chunk():
                row0 = cidx * cp_rows
                pltpu.sync_copy(racc.at[pl.ds(row0, cp_rows)], cp_v)
                pltpu.sync_copy(
                    cp_v, r_out.at[pl.ds(cid * n + row0, cp_rows)])

    return edge_pass


def kernel(x, edge_index, W, w):
    b, f_in, n = x.shape
    f = W.shape[0]
    e = edge_index.shape[1]
    ch = 80

    xT = x[0].T
    ii = edge_index[0]
    jj = edge_index[1]
    w_arr = jnp.asarray(w, jnp.float32).reshape(1, 1)

    u = _node_transform(xT, W.T)
    sp, qp = _make_stats(n, f, e, ch)(u, ii, jj)
    sp = sp.reshape(_NW, f)
    qp = qp.reshape(_NW, f)
    a2, b2 = _finalize(sp, qp, w_arr, float(e))
    fh = f // _NC
    u2 = jnp.concatenate([u[:, :fh], u[:, fh:]], axis=0)
    r_parts = _make_edge_pass(n, f, e, ch)(
        u2, ii, jj, a2.reshape(f), b2.reshape(f))
    r_full = jnp.concatenate([r_parts[:n], r_parts[n:]], axis=1)
    out = _node_output(r_full, W, w_arr)
    return out.T.reshape(1, f_in, n)
